# Initial kernel scaffold; baseline (speedup 1.0000x reference)
import jax
import jax.numpy as jnp
from jax.experimental import pallas as pl


def kernel(positions, neighbors_j, neighbors_k):
    raise NotImplementedError("write your pallas kernel here")


# speedup vs baseline: 369.2014x; 369.2014x over previous
import functools

import jax
import jax.numpy as jnp
from jax import lax
from jax.experimental import pallas as pl
from jax.experimental.pallas import tpu as pltpu
from jax.experimental.pallas import tpu_sc as plsc

_NC = 2
_NS = 16
_NW = _NC * _NS
_L = 16

_MAGIC = 0x5F3759DF


def _rsqrt(d2):
    seed = jnp.full((_L,), _MAGIC, jnp.int32) - (plsc.bitcast(d2, jnp.int32) >> 1)
    y = plsc.bitcast(seed, jnp.float32)
    half = d2 * 0.5
    y = y * (1.5 - half * y * y)
    y = y * (1.5 - half * y * y)
    return y


def _make_sc_kernel(B, N, T):
    BN = B * N
    rows_per = BN // _NW
    CH = 32
    n_chunks = rows_per // CH
    chunk_elems = CH * T

    mesh = plsc.VectorSubcoreMesh(
        core_axis_name="c", subcore_axis_name="s", num_cores=_NC,
        num_subcores=_NS)

    out_sds = jax.ShapeDtypeStruct((BN * T,), jnp.float32)

    @functools.partial(
        pl.kernel,
        out_type=(out_sds, out_sds, out_sds),
        mesh=mesh,
        compiler_params=pltpu.CompilerParams(needs_layout_passes=False),
        scratch_types=[
            pltpu.VMEM((BN,), jnp.float32),
            pltpu.VMEM((BN,), jnp.float32),
            pltpu.VMEM((BN,), jnp.float32),
            pltpu.VMEM((chunk_elems,), jnp.int32),
            pltpu.VMEM((chunk_elems,), jnp.int32),
            pltpu.VMEM((chunk_elems,), jnp.float32),
            pltpu.VMEM((chunk_elems,), jnp.float32),
            pltpu.VMEM((chunk_elems,), jnp.float32),
        ],
    )
    def sc_kernel(x_hbm, y_hbm, z_hbm, nj_hbm, nk_hbm,
                  rij_hbm, rik_hbm, rjk_hbm,
                  xv, yv, zv, njv, nkv, oij, oik, ojk):
        wid = lax.axis_index("s") * _NC + lax.axis_index("c")
        base_row = wid * rows_per
        batch_off = (base_row // N) * N

        pltpu.sync_copy(x_hbm, xv)
        pltpu.sync_copy(y_hbm, yv)
        pltpu.sync_copy(z_hbm, zv)

        def chunk_body(c, _):
            row0 = base_row + c * CH
            off = row0 * T
            pltpu.sync_copy(nj_hbm.at[pl.ds(off, chunk_elems)], njv)
            pltpu.sync_copy(nk_hbm.at[pl.ds(off, chunk_elems)], nkv)

            def row_body(r, _):
                row = row0 + r
                row_splat = jnp.full((_L,), row, jnp.int32)
                xi = plsc.load_gather(xv, [row_splat])
                yi = plsc.load_gather(yv, [row_splat])
                zi = plsc.load_gather(zv, [row_splat])
                for v in range(T // _L):
                    sl = pl.ds(r * T + v * _L, _L)
                    j = njv[sl] + batch_off
                    k = nkv[sl] + batch_off
                    xj = plsc.load_gather(xv, [j])
                    yj = plsc.load_gather(yv, [j])
                    zj = plsc.load_gather(zv, [j])
                    xk = plsc.load_gather(xv, [k])
                    yk = plsc.load_gather(yv, [k])
                    zk = plsc.load_gather(zv, [k])
                    dxij = xj - xi
                    dyij = yj - yi
                    dzij = zj - zi
                    dxik = xk - xi
                    dyik = yk - yi
                    dzik = zk - zi
                    dxjk = xj - xk
                    dyjk = yj - yk
                    dzjk = zj - zk
                    d2ij = dxij * dxij + dyij * dyij + dzij * dzij
                    d2ik = dxik * dxik + dyik * dyik + dzik * dzik
                    d2jk = dxjk * dxjk + dyjk * dyjk + dzjk * dzjk
                    oij[sl] = d2ij * _rsqrt(d2ij)
                    oik[sl] = d2ik * _rsqrt(d2ik)
                    ojk[sl] = d2jk * _rsqrt(d2jk)
                return 0

            lax.fori_loop(0, CH, row_body, 0)
            pltpu.sync_copy(oij, rij_hbm.at[pl.ds(off, chunk_elems)])
            pltpu.sync_copy(oik, rik_hbm.at[pl.ds(off, chunk_elems)])
            pltpu.sync_copy(ojk, rjk_hbm.at[pl.ds(off, chunk_elems)])
            return 0

        lax.fori_loop(0, n_chunks, chunk_body, 0)

    return sc_kernel


def kernel(positions, neighbors_j, neighbors_k):
    B, N, _ = positions.shape
    T = neighbors_j.shape[2]
    BN = B * N

    flat = positions.reshape(BN, 3)
    x = flat[:, 0].ravel()
    y = flat[:, 1].ravel()
    z = flat[:, 2].ravel()
    nj = neighbors_j.reshape(BN * T)
    nk = neighbors_k.reshape(BN * T)

    rij, rik, rjk = _make_sc_kernel(B, N, T)(x, y, z, nj, nk)
    shape = (B, N, T)
    return (rij.reshape(shape), rik.reshape(shape), rjk.reshape(shape))


# speedup vs baseline: 428.2650x; 1.1600x over previous
import functools

import jax
import jax.numpy as jnp
from jax import lax
from jax.experimental import pallas as pl
from jax.experimental.pallas import tpu as pltpu
from jax.experimental.pallas import tpu_sc as plsc

_NC = 2
_NS = 16
_NW = _NC * _NS
_L = 16

_MAGIC = 0x5F3759DF


def _rsqrt(d2):
    seed = jnp.full((_L,), _MAGIC, jnp.int32) - (plsc.bitcast(d2, jnp.int32) >> 1)
    y = plsc.bitcast(seed, jnp.float32)
    y = y * (1.5 - (0.5 * d2) * y * y)
    return y


def _make_sc_kernel(B, N, T):
    BN = B * N
    rows_per = BN // _NW
    CH = 32
    n_chunks = rows_per // CH
    chunk_elems = CH * T

    mesh = plsc.VectorSubcoreMesh(
        core_axis_name="c", subcore_axis_name="s", num_cores=_NC,
        num_subcores=_NS)

    out_sds = jax.ShapeDtypeStruct((BN * T,), jnp.float32)

    @functools.partial(
        pl.kernel,
        out_type=(out_sds, out_sds, out_sds),
        mesh=mesh,
        compiler_params=pltpu.CompilerParams(needs_layout_passes=False),
        scratch_types=[
            pltpu.VMEM((N,), jnp.float32),
            pltpu.VMEM((N,), jnp.float32),
            pltpu.VMEM((N,), jnp.float32),
            pltpu.VMEM((chunk_elems,), jnp.int32),
            pltpu.VMEM((chunk_elems,), jnp.int32),
            pltpu.VMEM((chunk_elems,), jnp.float32),
            pltpu.VMEM((chunk_elems,), jnp.float32),
            pltpu.VMEM((chunk_elems,), jnp.float32),
        ],
    )
    def sc_kernel(x_hbm, y_hbm, z_hbm, nj_hbm, nk_hbm,
                  rij_hbm, rik_hbm, rjk_hbm,
                  xv, yv, zv, njv, nkv, oij, oik, ojk):
        wid = lax.axis_index("s") * _NC + lax.axis_index("c")
        base_row = wid * rows_per
        batch_base = (base_row // N) * N
        base_local = base_row - batch_base

        pltpu.sync_copy(x_hbm.at[pl.ds(batch_base, N)], xv)
        pltpu.sync_copy(y_hbm.at[pl.ds(batch_base, N)], yv)
        pltpu.sync_copy(z_hbm.at[pl.ds(batch_base, N)], zv)

        def chunk_body(c, _):
            row0 = base_row + c * CH
            local0 = base_local + c * CH
            off = row0 * T
            pltpu.sync_copy(nj_hbm.at[pl.ds(off, chunk_elems)], njv)
            pltpu.sync_copy(nk_hbm.at[pl.ds(off, chunk_elems)], nkv)

            def row_body(r, _):
                row_splat = jnp.full((_L,), local0 + r, jnp.int32)
                xi = plsc.load_gather(xv, [row_splat])
                yi = plsc.load_gather(yv, [row_splat])
                zi = plsc.load_gather(zv, [row_splat])
                for v in range(T // _L):
                    sl = pl.ds(r * T + v * _L, _L)
                    j = njv[sl]
                    k = nkv[sl]
                    xj = plsc.load_gather(xv, [j])
                    yj = plsc.load_gather(yv, [j])
                    zj = plsc.load_gather(zv, [j])
                    xk = plsc.load_gather(xv, [k])
                    yk = plsc.load_gather(yv, [k])
                    zk = plsc.load_gather(zv, [k])
                    dxij = xj - xi
                    dyij = yj - yi
                    dzij = zj - zi
                    dxik = xk - xi
                    dyik = yk - yi
                    dzik = zk - zi
                    dxjk = xj - xk
                    dyjk = yj - yk
                    dzjk = zj - zk
                    d2ij = dxij * dxij + dyij * dyij + dzij * dzij
                    d2ik = dxik * dxik + dyik * dyik + dzik * dzik
                    d2jk = dxjk * dxjk + dyjk * dyjk + dzjk * dzjk
                    oij[sl] = d2ij * _rsqrt(d2ij)
                    oik[sl] = d2ik * _rsqrt(d2ik)
                    ojk[sl] = d2jk * _rsqrt(d2jk)
                return 0

            lax.fori_loop(0, CH, row_body, 0)
            pltpu.sync_copy(oij, rij_hbm.at[pl.ds(off, chunk_elems)])
            pltpu.sync_copy(oik, rik_hbm.at[pl.ds(off, chunk_elems)])
            pltpu.sync_copy(ojk, rjk_hbm.at[pl.ds(off, chunk_elems)])
            return 0

        lax.fori_loop(0, n_chunks, chunk_body, 0)

    return sc_kernel


def kernel(positions, neighbors_j, neighbors_k):
    B, N, _ = positions.shape
    T = neighbors_j.shape[2]
    BN = B * N

    flat = positions.reshape(BN, 3)
    x = flat[:, 0].ravel()
    y = flat[:, 1].ravel()
    z = flat[:, 2].ravel()
    nj = neighbors_j.reshape(BN * T)
    nk = neighbors_k.reshape(BN * T)

    rij, rik, rjk = _make_sc_kernel(B, N, T)(x, y, z, nj, nk)
    shape = (B, N, T)
    return (rij.reshape(shape), rik.reshape(shape), rjk.reshape(shape))


# speedup vs baseline: 548.4696x; 1.2807x over previous
import functools

import jax
import jax.numpy as jnp
from jax import lax
from jax.experimental import pallas as pl
from jax.experimental.pallas import tpu as pltpu
from jax.experimental.pallas import tpu_sc as plsc

_NC = 2
_NS = 16
_NW = _NC * _NS
_L = 16

_MAGIC = 0x5F3759DF


def _rsqrt(d2):
    seed = jnp.full((_L,), _MAGIC, jnp.int32) - (plsc.bitcast(d2, jnp.int32) >> 1)
    y = plsc.bitcast(seed, jnp.float32)
    y = y * (1.5 - (0.5 * d2) * y * y)
    return y


def _make_sc_kernel(B, N, T):
    BN = B * N
    rows_per = BN // _NW
    CH = 64
    n_chunks = rows_per // CH
    pairs = n_chunks // 2
    CE = CH * T

    mesh = plsc.VectorSubcoreMesh(
        core_axis_name="c", subcore_axis_name="s", num_cores=_NC,
        num_subcores=_NS)

    out_sds = jax.ShapeDtypeStruct((BN * T,), jnp.float32)

    @functools.partial(
        pl.kernel,
        out_type=(out_sds, out_sds, out_sds),
        mesh=mesh,
        compiler_params=pltpu.CompilerParams(needs_layout_passes=False),
        scratch_types=[
            pltpu.VMEM((N,), jnp.float32),
            pltpu.VMEM((N,), jnp.float32),
            pltpu.VMEM((N,), jnp.float32),
            pltpu.VMEM((CE,), jnp.int32),
            pltpu.VMEM((CE,), jnp.int32),
            pltpu.VMEM((CE,), jnp.int32),
            pltpu.VMEM((CE,), jnp.int32),
            pltpu.VMEM((CE,), jnp.float32),
            pltpu.VMEM((CE,), jnp.float32),
            pltpu.VMEM((CE,), jnp.float32),
            pltpu.VMEM((CE,), jnp.float32),
            pltpu.VMEM((CE,), jnp.float32),
            pltpu.VMEM((CE,), jnp.float32),
            pltpu.SemaphoreType.DMA,
            pltpu.SemaphoreType.DMA,
            pltpu.SemaphoreType.DMA,
            pltpu.SemaphoreType.DMA,
        ],
    )
    def sc_kernel(x_hbm, y_hbm, z_hbm, nj_hbm, nk_hbm,
                  rij_hbm, rik_hbm, rjk_hbm,
                  xv, yv, zv, nj_a, nj_b, nk_a, nk_b,
                  oij_a, oij_b, oik_a, oik_b, ojk_a, ojk_b,
                  sem_in_a, sem_in_b, sem_out_a, sem_out_b):
        wid = lax.axis_index("s") * _NC + lax.axis_index("c")
        base_row = wid * rows_per
        batch_base = (base_row // N) * N
        base_local = base_row - batch_base

        bufs = {
            0: (nj_a, nk_a, oij_a, oik_a, ojk_a, sem_in_a, sem_out_a),
            1: (nj_b, nk_b, oij_b, oik_b, ojk_b, sem_in_b, sem_out_b),
        }

        def start_in(c, p):
            njx, nkx, _, _, _, sem, _ = bufs[p]
            off = (base_row + c * CH) * T
            pltpu.async_copy(nj_hbm.at[pl.ds(off, CE)], njx, sem)
            pltpu.async_copy(nk_hbm.at[pl.ds(off, CE)], nkx, sem)

        def wait_in(c, p):
            njx, nkx, _, _, _, sem, _ = bufs[p]
            off = (base_row + c * CH) * T
            pltpu.make_async_copy(nj_hbm.at[pl.ds(off, CE)], njx, sem).wait()
            pltpu.make_async_copy(nk_hbm.at[pl.ds(off, CE)], nkx, sem).wait()

        def start_out(c, p):
            _, _, oij, oik, ojk, _, sem = bufs[p]
            off = (base_row + c * CH) * T
            pltpu.async_copy(oij, rij_hbm.at[pl.ds(off, CE)], sem)
            pltpu.async_copy(oik, rik_hbm.at[pl.ds(off, CE)], sem)
            pltpu.async_copy(ojk, rjk_hbm.at[pl.ds(off, CE)], sem)

        def wait_out(c, p):
            _, _, oij, oik, ojk, _, sem = bufs[p]
            off = (base_row + c * CH) * T
            pltpu.make_async_copy(oij, rij_hbm.at[pl.ds(off, CE)], sem).wait()
            pltpu.make_async_copy(oik, rik_hbm.at[pl.ds(off, CE)], sem).wait()
            pltpu.make_async_copy(ojk, rjk_hbm.at[pl.ds(off, CE)], sem).wait()

        def compute(c, p):
            njx, nkx, oij, oik, ojk, _, _ = bufs[p]
            local0 = base_local + c * CH

            def row_body(r, _):
                row_splat = jnp.full((_L,), local0 + r, jnp.int32)
                xi = plsc.load_gather(xv, [row_splat])
                yi = plsc.load_gather(yv, [row_splat])
                zi = plsc.load_gather(zv, [row_splat])
                for v in range(T // _L):
                    sl = pl.ds(r * T + v * _L, _L)
                    j = njx[sl]
                    k = nkx[sl]
                    xj = plsc.load_gather(xv, [j])
                    yj = plsc.load_gather(yv, [j])
                    zj = plsc.load_gather(zv, [j])
                    xk = plsc.load_gather(xv, [k])
                    yk = plsc.load_gather(yv, [k])
                    zk = plsc.load_gather(zv, [k])
                    dxij = xj - xi
                    dyij = yj - yi
                    dzij = zj - zi
                    dxik = xk - xi
                    dyik = yk - yi
                    dzik = zk - zi
                    dxjk = xj - xk
                    dyjk = yj - yk
                    dzjk = zj - zk
                    d2ij = dxij * dxij + dyij * dyij + dzij * dzij
                    d2ik = dxik * dxik + dyik * dyik + dzik * dzik
                    d2jk = dxjk * dxjk + dyjk * dyjk + dzjk * dzjk
                    oij[sl] = d2ij * _rsqrt(d2ij)
                    oik[sl] = d2ik * _rsqrt(d2ik)
                    ojk[sl] = d2jk * _rsqrt(d2jk)
                return 0

            lax.fori_loop(0, CH, row_body, 0)

        pltpu.sync_copy(x_hbm.at[pl.ds(batch_base, N)], xv)
        pltpu.sync_copy(y_hbm.at[pl.ds(batch_base, N)], yv)
        pltpu.sync_copy(z_hbm.at[pl.ds(batch_base, N)], zv)

        start_in(0, 0)

        def pair_body(c2, _):
            ca = 2 * c2
            cb = ca + 1
            start_in(cb, 1)
            wait_in(ca, 0)

            @pl.when(c2 > 0)
            def _():
                wait_out(ca - 2, 0)

            compute(ca, 0)
            start_out(ca, 0)

            @pl.when(c2 + 1 < pairs)
            def _():
                start_in(ca + 2, 0)

            wait_in(cb, 1)

            @pl.when(c2 > 0)
            def _():
                wait_out(cb - 2, 1)

            compute(cb, 1)
            start_out(cb, 1)
            return 0

        lax.fori_loop(0, pairs, pair_body, 0)
        wait_out(n_chunks - 2, 0)
        wait_out(n_chunks - 1, 1)

    return sc_kernel


def kernel(positions, neighbors_j, neighbors_k):
    B, N, _ = positions.shape
    T = neighbors_j.shape[2]
    BN = B * N

    flat = positions.reshape(BN, 3)
    x = flat[:, 0].ravel()
    y = flat[:, 1].ravel()
    z = flat[:, 2].ravel()
    nj = neighbors_j.reshape(BN * T)
    nk = neighbors_k.reshape(BN * T)

    rij, rik, rjk = _make_sc_kernel(B, N, T)(x, y, z, nj, nk)
    shape = (B, N, T)
    return (rij.reshape(shape), rik.reshape(shape), rjk.reshape(shape))


# speedup vs baseline: 778.9114x; 1.4202x over previous
import functools

import jax
import jax.numpy as jnp
from jax import lax
from jax.experimental import pallas as pl
from jax.experimental.pallas import tpu as pltpu
from jax.experimental.pallas import tpu_sc as plsc

_NC = 2
_NS = 16
_NW = _NC * _NS
_L = 16

_MAGIC = 0x5F3759DF


def _rsqrt(d2):
    seed = jnp.full((_L,), _MAGIC, jnp.int32) - (plsc.bitcast(d2, jnp.int32) >> 1)
    y = plsc.bitcast(seed, jnp.float32)
    y = y * (1.5 - (0.5 * d2) * y * y)
    return y


def _make_sc_kernel(B, N, T):
    BN = B * N
    rows_per = BN // _NW
    CH = 64
    n_chunks = rows_per // CH
    pairs = n_chunks // 2
    CE = CH * T

    mesh = plsc.VectorSubcoreMesh(
        core_axis_name="c", subcore_axis_name="s", num_cores=_NC,
        num_subcores=_NS)

    out_sds = jax.ShapeDtypeStruct((BN * T,), jnp.float32)

    @functools.partial(
        pl.kernel,
        out_type=(out_sds, out_sds, out_sds),
        mesh=mesh,
        compiler_params=pltpu.CompilerParams(needs_layout_passes=False),
        scratch_types=[
            pltpu.VMEM((N,), jnp.float32),
            pltpu.VMEM((N,), jnp.float32),
            pltpu.VMEM((N,), jnp.float32),
            pltpu.VMEM((CE,), jnp.int32),
            pltpu.VMEM((CE,), jnp.int32),
            pltpu.VMEM((CE,), jnp.int32),
            pltpu.VMEM((CE,), jnp.int32),
            pltpu.VMEM((CE,), jnp.float32),
            pltpu.VMEM((CE,), jnp.float32),
            pltpu.VMEM((CE,), jnp.float32),
            pltpu.VMEM((CE,), jnp.float32),
            pltpu.VMEM((CE,), jnp.float32),
            pltpu.VMEM((CE,), jnp.float32),
            pltpu.SemaphoreType.DMA,
            pltpu.SemaphoreType.DMA,
            pltpu.SemaphoreType.DMA,
            pltpu.SemaphoreType.DMA,
        ],
    )
    def sc_kernel(x_hbm, y_hbm, z_hbm, nj_hbm, nk_hbm,
                  rij_hbm, rik_hbm, rjk_hbm,
                  xv, yv, zv, nj_a, nj_b, nk_a, nk_b,
                  oij_a, oij_b, oik_a, oik_b, ojk_a, ojk_b,
                  sem_in_a, sem_in_b, sem_out_a, sem_out_b):
        wid = lax.axis_index("s") * _NC + lax.axis_index("c")
        base_row = wid * rows_per
        batch_base = (base_row // N) * N
        base_local = base_row - batch_base

        bufs = {
            0: (nj_a, nk_a, oij_a, oik_a, ojk_a, sem_in_a, sem_out_a),
            1: (nj_b, nk_b, oij_b, oik_b, ojk_b, sem_in_b, sem_out_b),
        }

        def start_in(c, p):
            njx, nkx, _, _, _, sem, _ = bufs[p]
            off = (base_row + c * CH) * T
            pltpu.async_copy(nj_hbm.at[pl.ds(off, CE)], njx, sem)
            pltpu.async_copy(nk_hbm.at[pl.ds(off, CE)], nkx, sem)

        def wait_in(c, p):
            njx, nkx, _, _, _, sem, _ = bufs[p]
            off = (base_row + c * CH) * T
            pltpu.make_async_copy(nj_hbm.at[pl.ds(off, CE)], njx, sem).wait()
            pltpu.make_async_copy(nk_hbm.at[pl.ds(off, CE)], nkx, sem).wait()

        def start_out(c, p):
            _, _, oij, oik, ojk, _, sem = bufs[p]
            off = (base_row + c * CH) * T
            pltpu.async_copy(oij, rij_hbm.at[pl.ds(off, CE)], sem)
            pltpu.async_copy(oik, rik_hbm.at[pl.ds(off, CE)], sem)
            pltpu.async_copy(ojk, rjk_hbm.at[pl.ds(off, CE)], sem)

        def wait_out(c, p):
            _, _, oij, oik, ojk, _, sem = bufs[p]
            off = (base_row + c * CH) * T
            pltpu.make_async_copy(oij, rij_hbm.at[pl.ds(off, CE)], sem).wait()
            pltpu.make_async_copy(oik, rik_hbm.at[pl.ds(off, CE)], sem).wait()
            pltpu.make_async_copy(ojk, rjk_hbm.at[pl.ds(off, CE)], sem).wait()

        def compute(c, p):
            njx, nkx, oij, oik, ojk, _, _ = bufs[p]
            local0 = base_local + c * CH

            @plsc.parallel_loop(0, CH, step=1, unroll=2)
            def row_body(r):
                row_splat = jnp.full((_L,), local0 + r, jnp.int32)
                xi = plsc.load_gather(xv, [row_splat])
                yi = plsc.load_gather(yv, [row_splat])
                zi = plsc.load_gather(zv, [row_splat])
                for v in range(T // _L):
                    sl = pl.ds(r * T + v * _L, _L)
                    j = njx[sl]
                    k = nkx[sl]
                    xj = plsc.load_gather(xv, [j])
                    yj = plsc.load_gather(yv, [j])
                    zj = plsc.load_gather(zv, [j])
                    xk = plsc.load_gather(xv, [k])
                    yk = plsc.load_gather(yv, [k])
                    zk = plsc.load_gather(zv, [k])
                    dxij = xj - xi
                    dyij = yj - yi
                    dzij = zj - zi
                    dxik = xk - xi
                    dyik = yk - yi
                    dzik = zk - zi
                    dxjk = xj - xk
                    dyjk = yj - yk
                    dzjk = zj - zk
                    d2ij = dxij * dxij + dyij * dyij + dzij * dzij
                    d2ik = dxik * dxik + dyik * dyik + dzik * dzik
                    d2jk = dxjk * dxjk + dyjk * dyjk + dzjk * dzjk
                    oij[sl] = d2ij * _rsqrt(d2ij)
                    oik[sl] = d2ik * _rsqrt(d2ik)
                    ojk[sl] = d2jk * _rsqrt(d2jk)

        pltpu.sync_copy(x_hbm.at[pl.ds(batch_base, N)], xv)
        pltpu.sync_copy(y_hbm.at[pl.ds(batch_base, N)], yv)
        pltpu.sync_copy(z_hbm.at[pl.ds(batch_base, N)], zv)

        start_in(0, 0)

        def pair_body(c2, _):
            ca = 2 * c2
            cb = ca + 1
            start_in(cb, 1)
            wait_in(ca, 0)

            @pl.when(c2 > 0)
            def _():
                wait_out(ca - 2, 0)

            compute(ca, 0)
            start_out(ca, 0)

            @pl.when(c2 + 1 < pairs)
            def _():
                start_in(ca + 2, 0)

            wait_in(cb, 1)

            @pl.when(c2 > 0)
            def _():
                wait_out(cb - 2, 1)

            compute(cb, 1)
            start_out(cb, 1)
            return 0

        lax.fori_loop(0, pairs, pair_body, 0)
        wait_out(n_chunks - 2, 0)
        wait_out(n_chunks - 1, 1)

    return sc_kernel


def kernel(positions, neighbors_j, neighbors_k):
    B, N, _ = positions.shape
    T = neighbors_j.shape[2]
    BN = B * N

    flat = positions.reshape(BN, 3)
    x = flat[:, 0].ravel()
    y = flat[:, 1].ravel()
    z = flat[:, 2].ravel()
    nj = neighbors_j.reshape(BN * T)
    nk = neighbors_k.reshape(BN * T)

    rij, rik, rjk = _make_sc_kernel(B, N, T)(x, y, z, nj, nk)
    shape = (B, N, T)
    return (rij.reshape(shape), rik.reshape(shape), rjk.reshape(shape))


# speedup vs baseline: 794.5910x; 1.0201x over previous
import functools

import jax
import jax.numpy as jnp
from jax import lax
from jax.experimental import pallas as pl
from jax.experimental.pallas import tpu as pltpu
from jax.experimental.pallas import tpu_sc as plsc

_NC = 2
_NS = 16
_NW = _NC * _NS
_L = 16

_MAGIC = 0x5F3759DF


def _dist(d2):
    seed = jnp.full((_L,), _MAGIC, jnp.int32) - (plsc.bitcast(d2, jnp.int32) >> 1)
    y = plsc.bitcast(seed, jnp.float32)
    t = d2 * y
    return t * (1.5 - 0.5 * (t * y))


def _make_sc_kernel(B, N, T):
    BN = B * N
    rows_per = BN // _NW
    CH = 64
    n_chunks = rows_per // CH
    pairs = n_chunks // 2
    CE = CH * T

    mesh = plsc.VectorSubcoreMesh(
        core_axis_name="c", subcore_axis_name="s", num_cores=_NC,
        num_subcores=_NS)

    out_sds = jax.ShapeDtypeStruct((BN * T,), jnp.float32)

    @functools.partial(
        pl.kernel,
        out_type=(out_sds, out_sds, out_sds),
        mesh=mesh,
        compiler_params=pltpu.CompilerParams(needs_layout_passes=False),
        scratch_types=[
            pltpu.VMEM((N,), jnp.float32),
            pltpu.VMEM((N,), jnp.float32),
            pltpu.VMEM((N,), jnp.float32),
            pltpu.VMEM((CE,), jnp.int32),
            pltpu.VMEM((CE,), jnp.int32),
            pltpu.VMEM((CE,), jnp.int32),
            pltpu.VMEM((CE,), jnp.int32),
            pltpu.VMEM((CE,), jnp.float32),
            pltpu.VMEM((CE,), jnp.float32),
            pltpu.VMEM((CE,), jnp.float32),
            pltpu.VMEM((CE,), jnp.float32),
            pltpu.VMEM((CE,), jnp.float32),
            pltpu.VMEM((CE,), jnp.float32),
            pltpu.SemaphoreType.DMA,
            pltpu.SemaphoreType.DMA,
            pltpu.SemaphoreType.DMA,
            pltpu.SemaphoreType.DMA,
        ],
    )
    def sc_kernel(x_hbm, y_hbm, z_hbm, nj_hbm, nk_hbm,
                  rij_hbm, rik_hbm, rjk_hbm,
                  xv, yv, zv, nj_a, nj_b, nk_a, nk_b,
                  oij_a, oij_b, oik_a, oik_b, ojk_a, ojk_b,
                  sem_in_a, sem_in_b, sem_out_a, sem_out_b):
        wid = lax.axis_index("s") * _NC + lax.axis_index("c")
        base_row = wid * rows_per
        batch_base = (base_row // N) * N
        base_local = base_row - batch_base

        bufs = {
            0: (nj_a, nk_a, oij_a, oik_a, ojk_a, sem_in_a, sem_out_a),
            1: (nj_b, nk_b, oij_b, oik_b, ojk_b, sem_in_b, sem_out_b),
        }

        def start_in(c, p):
            njx, nkx, _, _, _, sem, _ = bufs[p]
            off = (base_row + c * CH) * T
            pltpu.async_copy(nj_hbm.at[pl.ds(off, CE)], njx, sem)
            pltpu.async_copy(nk_hbm.at[pl.ds(off, CE)], nkx, sem)

        def wait_in(c, p):
            njx, nkx, _, _, _, sem, _ = bufs[p]
            off = (base_row + c * CH) * T
            pltpu.make_async_copy(nj_hbm.at[pl.ds(off, CE)], njx, sem).wait()
            pltpu.make_async_copy(nk_hbm.at[pl.ds(off, CE)], nkx, sem).wait()

        def start_out(c, p):
            _, _, oij, oik, ojk, _, sem = bufs[p]
            off = (base_row + c * CH) * T
            pltpu.async_copy(oij, rij_hbm.at[pl.ds(off, CE)], sem)
            pltpu.async_copy(oik, rik_hbm.at[pl.ds(off, CE)], sem)
            pltpu.async_copy(ojk, rjk_hbm.at[pl.ds(off, CE)], sem)

        def wait_out(c, p):
            _, _, oij, oik, ojk, _, sem = bufs[p]
            off = (base_row + c * CH) * T
            pltpu.make_async_copy(oij, rij_hbm.at[pl.ds(off, CE)], sem).wait()
            pltpu.make_async_copy(oik, rik_hbm.at[pl.ds(off, CE)], sem).wait()
            pltpu.make_async_copy(ojk, rjk_hbm.at[pl.ds(off, CE)], sem).wait()

        def compute(c, p):
            njx, nkx, oij, oik, ojk, _, _ = bufs[p]
            local0 = base_local + c * CH

            @plsc.parallel_loop(0, CH, step=1, unroll=2)
            def row_body(r):
                row_splat = jnp.full((_L,), local0 + r, jnp.int32)
                xi = plsc.load_gather(xv, [row_splat])
                yi = plsc.load_gather(yv, [row_splat])
                zi = plsc.load_gather(zv, [row_splat])
                for v in range(T // _L):
                    sl = pl.ds(r * T + v * _L, _L)
                    j = njx[sl]
                    k = nkx[sl]
                    xj = plsc.load_gather(xv, [j])
                    yj = plsc.load_gather(yv, [j])
                    zj = plsc.load_gather(zv, [j])
                    xk = plsc.load_gather(xv, [k])
                    yk = plsc.load_gather(yv, [k])
                    zk = plsc.load_gather(zv, [k])
                    dxij = xj - xi
                    dyij = yj - yi
                    dzij = zj - zi
                    dxik = xk - xi
                    dyik = yk - yi
                    dzik = zk - zi
                    dxjk = xj - xk
                    dyjk = yj - yk
                    dzjk = zj - zk
                    d2ij = dxij * dxij + dyij * dyij + dzij * dzij
                    d2ik = dxik * dxik + dyik * dyik + dzik * dzik
                    d2jk = dxjk * dxjk + dyjk * dyjk + dzjk * dzjk
                    oij[sl] = _dist(d2ij)
                    oik[sl] = _dist(d2ik)
                    ojk[sl] = _dist(d2jk)

        pltpu.sync_copy(x_hbm.at[pl.ds(batch_base, N)], xv)
        pltpu.sync_copy(y_hbm.at[pl.ds(batch_base, N)], yv)
        pltpu.sync_copy(z_hbm.at[pl.ds(batch_base, N)], zv)

        start_in(0, 0)

        def pair_body(c2, _):
            ca = 2 * c2
            cb = ca + 1
            start_in(cb, 1)
            wait_in(ca, 0)

            @pl.when(c2 > 0)
            def _():
                wait_out(ca - 2, 0)

            compute(ca, 0)
            start_out(ca, 0)

            @pl.when(c2 + 1 < pairs)
            def _():
                start_in(ca + 2, 0)

            wait_in(cb, 1)

            @pl.when(c2 > 0)
            def _():
                wait_out(cb - 2, 1)

            compute(cb, 1)
            start_out(cb, 1)
            return 0

        lax.fori_loop(0, pairs, pair_body, 0)
        wait_out(n_chunks - 2, 0)
        wait_out(n_chunks - 1, 1)

    return sc_kernel


def kernel(positions, neighbors_j, neighbors_k):
    B, N, _ = positions.shape
    T = neighbors_j.shape[2]
    BN = B * N

    flat = positions.reshape(BN, 3)
    x = flat[:, 0].ravel()
    y = flat[:, 1].ravel()
    z = flat[:, 2].ravel()
    nj = neighbors_j.reshape(BN * T)
    nk = neighbors_k.reshape(BN * T)

    rij, rik, rjk = _make_sc_kernel(B, N, T)(x, y, z, nj, nk)
    shape = (B, N, T)
    return (rij.reshape(shape), rik.reshape(shape), rjk.reshape(shape))
